# baseline (speedup 1.0000x reference)
import functools

import jax
import jax.numpy as jnp
from jax.experimental import pallas as pl
from jax.experimental.pallas import tpu as pltpu

N = 10000
D = 128
BM = 200


def _sage_kernel(feat_ref, adja_ref, adjb_ref, feats_ref, w1_ref, w2_ref,
                 out_ref):
    feats = feats_ref[...]
    adj = jnp.concatenate([adja_ref[...], adjb_ref[...]], axis=0)
    rowsum = jnp.sum(adj, axis=1, keepdims=True)
    neigh = jnp.dot(adj, feats, preferred_element_type=jnp.float32)
    scale = 1.0 / (rowsum + 1.0)
    out_ref[...] = (
        jnp.dot(feat_ref[...], w1_ref[...], preferred_element_type=jnp.float32)
        + jnp.dot(neigh * scale, w2_ref[...], preferred_element_type=jnp.float32)
    )


@functools.partial(jax.jit, static_argnames=())
def kernel(features, adj, W):
    w1 = W[:, :D].T
    w2 = W[:, D:].T
    grid = (N // (2 * BM),)
    return pl.pallas_call(
        _sage_kernel,
        grid=grid,
        in_specs=[
            pl.BlockSpec((2 * BM, D), lambda i: (i, 0)),
            pl.BlockSpec((BM, N), lambda i: (2 * i, 0)),
            pl.BlockSpec((BM, N), lambda i: (2 * i + 1, 0)),
            pl.BlockSpec((N, D), lambda i: (0, 0)),
            pl.BlockSpec((D, D), lambda i: (0, 0)),
            pl.BlockSpec((D, D), lambda i: (0, 0)),
        ],
        out_specs=pl.BlockSpec((2 * BM, D), lambda i: (i, 0)),
        out_shape=jax.ShapeDtypeStruct((N, D), jnp.float32),
        compiler_params=pltpu.CompilerParams(
            dimension_semantics=("arbitrary",),
        ),
    )(features, adj, adj, features, w1, w2)
